# Initial kernel scaffold; baseline (speedup 1.0000x reference)
import jax
import jax.numpy as jnp
from jax.experimental import pallas as pl


def kernel(x, token_table, segment_table, position_table):
    raise NotImplementedError("write your pallas kernel here")


# speedup vs baseline: 3.1321x; 3.1321x over previous
import functools

import jax
import jax.numpy as jnp
from jax import lax
from jax.experimental import pallas as pl
from jax.experimental.pallas import tpu as pltpu
from jax.experimental.pallas import tpu_sc as plsc

_B = 1024
_INPUT = 511
_D = 128
_VREGS_PER_ROW = _D // 16
_P = 128
_NC = 2
_NS = 16
_NW = _NC * _NS
_ROWS_PER_W = _B // _NW


def _sc_body(x_ref, tok_ref, seg_ref, pos_ref, out_ref,
             bias_v, seg_v, idx_v, row_v, sem):
    wid = lax.axis_index("s") * _NC + lax.axis_index("c")

    pltpu.sync_copy(pos_ref, bias_v)
    pltpu.sync_copy(seg_ref, seg_v)

    def _add_seg(lo, hi, seg_row):
        seg_regs = [seg_v[seg_row, pl.ds(16 * j, 16)]
                    for j in range(_VREGS_PER_ROW)]

        def body(i, carry):
            for j in range(_VREGS_PER_ROW):
                sl = pl.ds(16 * j, 16)
                bias_v[i, sl] = bias_v[i, sl] + seg_regs[j]
            return carry

        lax.fori_loop(lo, hi, body, 0)

    _add_seg(0, 256, 0)
    _add_seg(256, 512, 1)

    b0 = wid * _ROWS_PER_W

    def row_body(b, carry):
        for c in range(4):
            p0 = c * _P
            n_store = _P if c < 3 else _INPUT - 3 * _P
            pltpu.sync_copy(x_ref.at[b, pl.ds(p0, _P)], idx_v)
            pltpu.async_copy(tok_ref.at[idx_v], row_v, sem).wait()

            def add_body(i, inner_carry):
                for j in range(_VREGS_PER_ROW):
                    sl = pl.ds(16 * j, 16)
                    row_v[i, sl] = row_v[i, sl] + bias_v[p0 + i, sl]
                return inner_carry

            lax.fori_loop(0, _P, add_body, 0)
            pltpu.sync_copy(row_v.at[pl.ds(0, n_store)],
                            out_ref.at[b, pl.ds(p0, n_store)])
        return carry

    lax.fori_loop(b0, b0 + _ROWS_PER_W, row_body, 0)


@functools.partial(jax.jit, donate_argnums=())
def _run(x_pad, token_table, segment_table, position_table):
    mesh = plsc.VectorSubcoreMesh(core_axis_name="c", subcore_axis_name="s")
    fn = functools.partial(
        pl.kernel,
        mesh=mesh,
        out_type=jax.ShapeDtypeStruct((_B, _INPUT, _D), jnp.float32),
        scratch_types=[
            pltpu.VMEM((_INPUT + 1, _D), jnp.float32),
            pltpu.VMEM((2, _D), jnp.float32),
            pltpu.VMEM((_P,), jnp.int32),
            pltpu.VMEM((_P, _D), jnp.float32),
            pltpu.SemaphoreType.DMA,
        ],
    )(_sc_body)
    return fn(x_pad, token_table, segment_table, position_table)


def kernel(x, token_table, segment_table, position_table):
    x_pad = jnp.pad(x, ((0, 0), (0, 1)))
    return _run(x_pad, token_table, segment_table, position_table)


# speedup vs baseline: 4.6971x; 1.4997x over previous
import functools

import jax
import jax.numpy as jnp
from jax import lax
from jax.experimental import pallas as pl
from jax.experimental.pallas import tpu as pltpu
from jax.experimental.pallas import tpu_sc as plsc

_B = 1024
_INPUT = 511
_D = 128
_NVR = _D // 16
_NC = 2
_NS = 16
_NW = _NC * _NS
_PW = 16
_CB = 128
_NBB = _B // _CB
_NCHUNK = _PW * _NBB
_NSLOT = 4


def _sc_body(xt_ref, tok_ref, seg_ref, pos_ref, out_ref,
             idx_v, bias_v, seg_v, rows, gsems, ssems):
    w = lax.axis_index("s") * _NC + lax.axis_index("c")
    p_base = w * _PW

    pltpu.sync_copy(xt_ref.at[pl.ds(p_base, _PW)], idx_v)

    pltpu.sync_copy(pos_ref.at[pl.ds(p_base, _PW)], bias_v)
    pltpu.sync_copy(seg_ref, seg_v)
    seg_row = jnp.where(w >= 16, 1, 0)
    for i in range(_PW):
        for j in range(_NVR):
            sl = pl.ds(16 * j, 16)
            bias_v[i, sl] = bias_v[i, sl] + seg_v[seg_row, sl]

    def start_gather(t, s):
        pi = t >> 3
        bb = t & 7
        pltpu.async_copy(tok_ref.at[idx_v.at[pi, bb]], rows[s], gsems[s])

    def wait_gather(s):
        pltpu.make_async_copy(tok_ref.at[pl.ds(0, _CB)], rows[s],
                              gsems[s]).wait()

    def start_store(t, s):
        pi = t >> 3
        b0 = (t & 7) * _CB
        p = p_base + pi

        @pl.when(p < _INPUT)
        def _():
            pltpu.async_copy(rows[s], out_ref.at[pl.ds(b0, _CB), p],
                             ssems[s])

    def wait_store(t, s):
        p = p_base + (t >> 3)

        @pl.when(p < _INPUT)
        def _():
            pltpu.make_async_copy(tok_ref.at[pl.ds(0, _CB)], rows[s],
                                  ssems[s]).wait()

    def add_bias(t, s):
        pi = t >> 3
        row_v = rows[s]
        bias_regs = [bias_v[pi, pl.ds(16 * j, 16)] for j in range(_NVR)]

        def body(i, carry):
            for u in range(4):
                r = 4 * i + u
                for j in range(_NVR):
                    sl = pl.ds(16 * j, 16)
                    row_v[r, sl] = row_v[r, sl] + bias_regs[j]
            return carry

        lax.fori_loop(0, _CB // 4, body, 0)

    start_gather(0, 0)
    start_gather(1, 1)

    def ring_body(i, carry):
        for u in range(_NSLOT):
            t = _NSLOT * i + u
            s = u
            wait_gather(s)
            add_bias(t, s)
            start_store(t, s)
            s2 = (u + 2) % _NSLOT

            @pl.when(t >= 2)
            def _():
                wait_store(t - 2, s2)

            @pl.when(t + 2 < _NCHUNK)
            def _():
                start_gather(t + 2, s2)
        return carry

    lax.fori_loop(0, _NCHUNK // _NSLOT, ring_body, 0)

    wait_store(_NCHUNK - 2, (_NCHUNK - 2) % _NSLOT)
    wait_store(_NCHUNK - 1, (_NCHUNK - 1) % _NSLOT)


@jax.jit
def _run(x_t3, token_table, segment_table, position_table):
    mesh = plsc.VectorSubcoreMesh(core_axis_name="c", subcore_axis_name="s")
    fn = functools.partial(
        pl.kernel,
        mesh=mesh,
        out_type=jax.ShapeDtypeStruct((_B, _INPUT, _D), jnp.float32),
        scratch_types=[
            pltpu.VMEM((_PW, _NBB, _CB), jnp.int32),
            pltpu.VMEM((_PW, _D), jnp.float32),
            pltpu.VMEM((2, _D), jnp.float32),
            [pltpu.VMEM((_CB, _D), jnp.float32) for _ in range(_NSLOT)],
            [pltpu.SemaphoreType.DMA for _ in range(_NSLOT)],
            [pltpu.SemaphoreType.DMA for _ in range(_NSLOT)],
        ],
    )(_sc_body)
    return fn(x_t3, token_table, segment_table, position_table)


def kernel(x, token_table, segment_table, position_table):
    x_pad = jnp.pad(x, ((0, 0), (0, 1)))
    x_t3 = x_pad.T.reshape(_INPUT + 1, _NBB, _CB)
    return _run(x_t3, token_table, segment_table, position_table)


# speedup vs baseline: 8.2968x; 1.7664x over previous
import functools

import jax
import jax.numpy as jnp
from jax import lax
from jax.experimental import pallas as pl
from jax.experimental.pallas import tpu as pltpu
from jax.experimental.pallas import tpu_sc as plsc

_B = 1024
_INPUT = 511
_D = 128
_NVR = _D // 16
_NC = 2
_NS = 16
_NW = _NC * _NS
_PW = 16
_CB = 128
_NBB = _B // _CB
_NCHUNK = _PW * _NBB
_NSLOT = 4


def _sc_body(xt_ref, tok_ref, seg_ref, pos_ref, out_ref,
             idx_v, bias_v, seg_v, rows, gsems, ssems):
    w = lax.axis_index("s") * _NC + lax.axis_index("c")
    p_base = w * _PW

    pltpu.sync_copy(xt_ref.at[pl.ds(p_base, _PW)], idx_v)

    pltpu.sync_copy(pos_ref.at[pl.ds(p_base, _PW)], bias_v)
    pltpu.sync_copy(seg_ref, seg_v)
    seg_row = jnp.where(w >= 16, 1, 0)
    for i in range(_PW):
        for j in range(_NVR):
            sl = pl.ds(16 * j, 16)
            bias_v[i, sl] = bias_v[i, sl] + seg_v[seg_row, sl]

    def start_gather(t, s):
        pi = t >> 3
        bb = t & 7
        pltpu.async_copy(tok_ref.at[idx_v.at[pi, bb]], rows[s], gsems[s])

    def wait_gather(s):
        pltpu.make_async_copy(tok_ref.at[pl.ds(0, _CB)], rows[s],
                              gsems[s]).wait()

    def start_store(t, s):
        pi = t >> 3
        b0 = (t & 7) * _CB
        p = p_base + pi

        @pl.when(p < _INPUT)
        def _():
            pltpu.async_copy(rows[s], out_ref.at[p, pl.ds(b0, _CB)],
                             ssems[s])

    def wait_store(t, s):
        p = p_base + (t >> 3)

        @pl.when(p < _INPUT)
        def _():
            pltpu.make_async_copy(tok_ref.at[pl.ds(0, _CB)], rows[s],
                                  ssems[s]).wait()

    def add_bias(t, s):
        pi = t >> 3
        row_v = rows[s]
        bias_regs = [bias_v[pi, pl.ds(16 * j, 16)] for j in range(_NVR)]

        def body(i, carry):
            for u in range(4):
                r = 4 * i + u
                for j in range(_NVR):
                    sl = pl.ds(16 * j, 16)
                    row_v[r, sl] = row_v[r, sl] + bias_regs[j]
            return carry

        lax.fori_loop(0, _CB // 4, body, 0)

    start_gather(0, 0)
    start_gather(1, 1)

    def ring_body(i, carry):
        for u in range(_NSLOT):
            t = _NSLOT * i + u
            s = u
            wait_gather(s)
            add_bias(t, s)
            start_store(t, s)
            s2 = (u + 2) % _NSLOT

            @pl.when(t >= 2)
            def _():
                wait_store(t - 2, s2)

            @pl.when(t + 2 < _NCHUNK)
            def _():
                start_gather(t + 2, s2)
        return carry

    lax.fori_loop(0, _NCHUNK // _NSLOT, ring_body, 0)

    wait_store(_NCHUNK - 2, (_NCHUNK - 2) % _NSLOT)
    wait_store(_NCHUNK - 1, (_NCHUNK - 1) % _NSLOT)


@jax.jit
def _run(x_t3, token_table, segment_table, position_table):
    mesh = plsc.VectorSubcoreMesh(core_axis_name="c", subcore_axis_name="s")
    fn = functools.partial(
        pl.kernel,
        mesh=mesh,
        out_type=jax.ShapeDtypeStruct((_INPUT, _B, _D), jnp.float32),
        scratch_types=[
            pltpu.VMEM((_PW, _NBB, _CB), jnp.int32),
            pltpu.VMEM((_PW, _D), jnp.float32),
            pltpu.VMEM((2, _D), jnp.float32),
            [pltpu.VMEM((_CB, _D), jnp.float32) for _ in range(_NSLOT)],
            [pltpu.SemaphoreType.DMA for _ in range(_NSLOT)],
            [pltpu.SemaphoreType.DMA for _ in range(_NSLOT)],
        ],
    )(_sc_body)
    return fn(x_t3, token_table, segment_table, position_table)


def kernel(x, token_table, segment_table, position_table):
    x_pad = jnp.pad(x, ((0, 0), (0, 1)))
    x_t3 = x_pad.T.reshape(_INPUT + 1, _NBB, _CB)
    out_t = _run(x_t3, token_table, segment_table, position_table)
    return out_t.transpose(1, 0, 2)


# speedup vs baseline: 8.3114x; 1.0018x over previous
import functools

import jax
import jax.numpy as jnp
from jax import lax
from jax.experimental import pallas as pl
from jax.experimental.pallas import tpu as pltpu
from jax.experimental.pallas import tpu_sc as plsc

_B = 1024
_INPUT = 511
_D = 128
_NVR = _D // 16
_NC = 2
_NS = 16
_NW = _NC * _NS
_PW = 16
_CB = 128
_NBB = _B // _CB
_NCHUNK = _PW * _NBB
_NSLOT = 4


def _sc_body(xt_ref, tok_ref, seg_ref, pos_ref, out_ref,
             idx_v, bias_v, seg_v, rows, gsems, ssems):
    w = lax.axis_index("s") * _NC + lax.axis_index("c")
    p_base = w * _PW

    pltpu.sync_copy(xt_ref.at[pl.ds(p_base, _PW)], idx_v)

    pltpu.sync_copy(pos_ref.at[pl.ds(p_base, _PW)], bias_v)
    pltpu.sync_copy(seg_ref, seg_v)
    seg_row = jnp.where(w >= 16, 1, 0)
    for i in range(_PW):
        for j in range(_NVR):
            sl = pl.ds(16 * j, 16)
            bias_v[i, sl] = bias_v[i, sl] + seg_v[seg_row, sl]

    def start_gather(t, s):
        pi = t >> 3
        bb = t & 7
        pltpu.async_copy(tok_ref.at[idx_v.at[pi, bb]], rows[s], gsems[s])

    def wait_gather(s):
        pltpu.make_async_copy(tok_ref.at[pl.ds(0, _CB)], rows[s],
                              gsems[s]).wait()

    def start_store(t, s):
        pi = t >> 3
        b0 = (t & 7) * _CB
        p = p_base + pi

        @pl.when(p < _INPUT)
        def _():
            pltpu.async_copy(rows[s], out_ref.at[p, pl.ds(b0, _CB)],
                             ssems[s])

    def wait_store(t, s):
        p = p_base + (t >> 3)

        @pl.when(p < _INPUT)
        def _():
            pltpu.make_async_copy(tok_ref.at[pl.ds(0, _CB)], rows[s],
                                  ssems[s]).wait()

    def add_bias(t, s):
        pi = t >> 3
        row_v = rows[s]
        bias_regs = [bias_v[pi, pl.ds(16 * j, 16)] for j in range(_NVR)]

        def body(i, carry):
            for u in range(4):
                r = 4 * i + u
                for j in range(_NVR):
                    plsc.addupdate(row_v.at[r, pl.ds(16 * j, 16)],
                                   bias_regs[j])
            return carry

        lax.fori_loop(0, _CB // 4, body, 0)

    start_gather(0, 0)
    start_gather(1, 1)

    def ring_body(i, carry):
        for u in range(_NSLOT):
            t = _NSLOT * i + u
            s = u
            wait_gather(s)
            add_bias(t, s)
            start_store(t, s)
            s2 = (u + 2) % _NSLOT

            @pl.when(t >= 2)
            def _():
                wait_store(t - 2, s2)

            @pl.when(t + 2 < _NCHUNK)
            def _():
                start_gather(t + 2, s2)
        return carry

    lax.fori_loop(0, _NCHUNK // _NSLOT, ring_body, 0)

    wait_store(_NCHUNK - 2, (_NCHUNK - 2) % _NSLOT)
    wait_store(_NCHUNK - 1, (_NCHUNK - 1) % _NSLOT)


@jax.jit
def _run(x_t3, token_table, segment_table, position_table):
    mesh = plsc.VectorSubcoreMesh(core_axis_name="c", subcore_axis_name="s")
    fn = functools.partial(
        pl.kernel,
        mesh=mesh,
        out_type=jax.ShapeDtypeStruct((_INPUT, _B, _D), jnp.float32),
        scratch_types=[
            pltpu.VMEM((_PW, _NBB, _CB), jnp.int32),
            pltpu.VMEM((_PW, _D), jnp.float32),
            pltpu.VMEM((2, _D), jnp.float32),
            [pltpu.VMEM((_CB, _D), jnp.float32) for _ in range(_NSLOT)],
            [pltpu.SemaphoreType.DMA for _ in range(_NSLOT)],
            [pltpu.SemaphoreType.DMA for _ in range(_NSLOT)],
        ],
    )(_sc_body)
    return fn(x_t3, token_table, segment_table, position_table)


def kernel(x, token_table, segment_table, position_table):
    x_pad = jnp.pad(x, ((0, 0), (0, 1)))
    x_t3 = x_pad.T.reshape(_INPUT + 1, _NBB, _CB)
    out_t = _run(x_t3, token_table, segment_table, position_table)
    return out_t.transpose(1, 0, 2)
